# baseline (speedup 1.0000x reference)
import functools

import numpy as np
import jax
import jax.numpy as jnp
from jax.experimental import pallas as pl
from jax.experimental.pallas import tpu as pltpu

_TB = 128
_N = 32 * _TB


def _shl(v, k):
    if k == 0:
        return v
    return jnp.concatenate([v[:, k:], v[:, :k]], axis=1)


def _shu(v):
    return jnp.concatenate([v[1:, :], v[:1, :]], axis=0)


def _net_kernel(x_ref, a1_ref, b1_ref, a2_ref, b2_ref, af_ref, bf1_ref,
                w2_ref, bf2_ref, w3_ref, bf3_ref, out_ref):
    pieces = []
    for c in range(3):
        tc = x_ref[:, c, :].T
        pieces.append(tc.reshape(32, 32 * _TB))
    x = jnp.concatenate(pieces, axis=0)

    acc = None
    for dj in range(5):
        d = jnp.dot(a1_ref[dj], _shl(x, dj * _TB),
                    preferred_element_type=jnp.float32)
        acc = d if acc is None else acc + d
    y1 = jnp.maximum(acc + b1_ref[...], 0.0)

    t = jnp.maximum(y1, _shl(y1, _TB))
    p1 = jnp.maximum(t, _shu(t)).astype(jnp.bfloat16)

    acc = None
    for dj in range(5):
        d = jnp.dot(a2_ref[dj], _shl(p1, 2 * dj * _TB),
                    preferred_element_type=jnp.float32)
        acc = d if acc is None else acc + d
    y2 = jnp.maximum(acc + b2_ref[...], 0.0)

    t = jnp.maximum(y2, _shl(y2, 2 * _TB))
    p2 = jnp.maximum(t, _shu(t)).astype(jnp.bfloat16)

    acc = None
    for pw in range(5):
        blk = p2[:, 4 * pw * _TB:(4 * pw + 1) * _TB]
        d = jnp.dot(af_ref[pw], blk, preferred_element_type=jnp.float32)
        acc = d if acc is None else acc + d
    h = jnp.maximum(acc + bf1_ref[...], 0.0)

    ht = h.T.astype(jnp.bfloat16)
    h2 = jnp.maximum(jnp.dot(ht, w2_ref[...], preferred_element_type=jnp.float32)
                     + bf2_ref[...], 0.0).astype(jnp.bfloat16)
    out = jnp.dot(h2, w3_ref[...], preferred_element_type=jnp.float32) + bf3_ref[...]
    out_ref[...] = out


def _prep_params(conv1_w, conv1_b, conv2_w, conv2_b,
                 fc1_w, fc1_b, fc2_w, fc2_b, fc3_w, fc3_b):
    f32, bf16 = jnp.float32, jnp.bfloat16

    co, c, di, dj, i = np.meshgrid(np.arange(6), np.arange(3), np.arange(5),
                                   np.arange(5), np.arange(28), indexing='ij')
    v1 = jnp.broadcast_to(conv1_w.astype(f32)[:, :, :, :, None], (6, 3, 5, 5, 28))
    a1 = jnp.zeros((5, 168, 96), f32).at[
        dj.ravel(), (28 * co + i).ravel(), (32 * c + i + di).ravel()
    ].set(v1.reshape(-1)).astype(bf16)

    co2, c2, di2, dj2, i2 = np.meshgrid(np.arange(16), np.arange(6), np.arange(5),
                                        np.arange(5), np.arange(10), indexing='ij')
    v2 = jnp.broadcast_to(conv2_w.astype(f32)[:, :, :, :, None], (16, 6, 5, 5, 10))
    a2 = jnp.zeros((5, 160, 168), f32).at[
        dj2.ravel(), (10 * co2 + i2).ravel(), (28 * c2 + 2 * (i2 + di2)).ravel()
    ].set(v2.reshape(-1)).astype(bf16)

    o, fo, pp, pw = np.meshgrid(np.arange(32), np.arange(16), np.arange(5),
                                np.arange(5), indexing='ij')
    vf = fc1_w.astype(f32).reshape(32, 16, 5, 5)
    af = jnp.zeros((5, 32, 160), f32).at[
        pw.ravel(), o.ravel(), (10 * fo + 2 * pp).ravel()
    ].set(vf.reshape(-1)).astype(bf16)

    return dict(
        a1=a1, b1=jnp.repeat(conv1_b.astype(f32), 28).reshape(168, 1),
        a2=a2, b2=jnp.repeat(conv2_b.astype(f32), 10).reshape(160, 1),
        af=af, bf1=fc1_b.astype(f32).reshape(32, 1),
        w2=fc2_w.astype(bf16).T, bf2=fc2_b.astype(f32).reshape(1, 16),
        w3=fc3_w.astype(bf16).T, bf3=fc3_b.astype(f32).reshape(1, 10),
    )


def kernel(x, conv1_w, conv1_b, conv2_w, conv2_b,
           fc1_w, fc1_b, fc2_w, fc2_b, fc3_w, fc3_b):
    kp = _prep_params(conv1_w, conv1_b, conv2_w, conv2_b,
                      fc1_w, fc1_b, fc2_w, fc2_b, fc3_w, fc3_b)
    B = x.shape[0]
    Bp = ((B + _TB - 1) // _TB) * _TB
    xb = x.astype(jnp.bfloat16)
    if Bp != B:
        xb = jnp.pad(xb, ((0, Bp - B), (0, 0), (0, 0), (0, 0)))
    ntiles = Bp // _TB
    x3 = xb.reshape(Bp, 3, 1024)

    flops = int(Bp * 2 * (75 * 6 * 28 * 28 + 150 * 16 * 10 * 10
                          + 400 * 32 + 32 * 16 + 16 * 10))
    bytes_accessed = int(x3.size * 2 + Bp * 10 * 4
                         + sum(int(v.size) for v in kp.values()) * 2)

    out = pl.pallas_call(
        _net_kernel,
        out_shape=jax.ShapeDtypeStruct((Bp, 10), jnp.float32),
        grid=(ntiles,),
        in_specs=[
            pl.BlockSpec((_TB, 3, 1024), lambda i: (i, 0, 0)),
            pl.BlockSpec((5, 168, 96), lambda i: (0, 0, 0)),
            pl.BlockSpec((168, 1), lambda i: (0, 0)),
            pl.BlockSpec((5, 160, 168), lambda i: (0, 0, 0)),
            pl.BlockSpec((160, 1), lambda i: (0, 0)),
            pl.BlockSpec((5, 32, 160), lambda i: (0, 0, 0)),
            pl.BlockSpec((32, 1), lambda i: (0, 0)),
            pl.BlockSpec((32, 16), lambda i: (0, 0)),
            pl.BlockSpec((1, 16), lambda i: (0, 0)),
            pl.BlockSpec((16, 10), lambda i: (0, 0)),
            pl.BlockSpec((1, 10), lambda i: (0, 0)),
        ],
        out_specs=pl.BlockSpec((_TB, 10), lambda i: (i, 0)),
        compiler_params=pltpu.CompilerParams(dimension_semantics=("parallel",)),
        cost_estimate=pl.CostEstimate(flops=flops, transcendentals=0,
                                      bytes_accessed=bytes_accessed),
    )(x3, kp['a1'], kp['b1'], kp['a2'], kp['b2'], kp['af'], kp['bf1'],
      kp['w2'], kp['bf2'], kp['w3'], kp['bf3'])
    return out[:B]


# speedup vs baseline: 2.4911x; 2.4911x over previous
import functools

import numpy as np
import jax
import jax.numpy as jnp
from jax.experimental import pallas as pl
from jax.experimental.pallas import tpu as pltpu

_TB = 128
_N = 32 * _TB


def _shl(v, k):
    if k == 0:
        return v
    return jnp.concatenate([v[:, k:], v[:, :k]], axis=1)


def _shu(v):
    return jnp.concatenate([v[1:, :], v[:1, :]], axis=0)


def _net_kernel(x_ref, a1_ref, b1_ref, a2_ref, b2_ref, af_ref, bf1_ref,
                w2_ref, bf2_ref, w3_ref, bf3_ref, out_ref):
    pieces = []
    for c in range(3):
        tc = x_ref[:, c, :].T
        pieces.append(tc.reshape(32, 32 * _TB))
    x = jnp.concatenate(pieces, axis=0)

    acc = None
    for dj in range(5):
        d = jnp.dot(a1_ref[dj], _shl(x, dj * _TB),
                    preferred_element_type=jnp.float32)
        acc = d if acc is None else acc + d
    y1 = jnp.maximum(acc + b1_ref[...], 0.0)

    t = jnp.maximum(y1, _shl(y1, _TB))
    p1 = jnp.maximum(t, _shu(t)).astype(jnp.bfloat16)

    acc = None
    for dj in range(5):
        d = jnp.dot(a2_ref[dj], _shl(p1, 2 * dj * _TB),
                    preferred_element_type=jnp.float32)
        acc = d if acc is None else acc + d
    y2 = jnp.maximum(acc + b2_ref[...], 0.0)

    t = jnp.maximum(y2, _shl(y2, 2 * _TB))
    p2 = jnp.maximum(t, _shu(t)).astype(jnp.bfloat16)

    acc = None
    for pw in range(5):
        blk = p2[:, 4 * pw * _TB:(4 * pw + 1) * _TB]
        d = jnp.dot(af_ref[pw], blk, preferred_element_type=jnp.float32)
        acc = d if acc is None else acc + d
    h = jnp.maximum(acc + bf1_ref[...], 0.0)

    ht = h.T.astype(jnp.bfloat16)
    h2 = jnp.maximum(jnp.dot(ht, w2_ref[...], preferred_element_type=jnp.float32)
                     + bf2_ref[...], 0.0).astype(jnp.bfloat16)
    out = jnp.dot(h2, w3_ref[...], preferred_element_type=jnp.float32) + bf3_ref[...]
    out_ref[...] = out


def _prep_params(conv1_w, conv1_b, conv2_w, conv2_b,
                 fc1_w, fc1_b, fc2_w, fc2_b, fc3_w, fc3_b):
    f32, bf16 = jnp.float32, jnp.bfloat16


    e1 = np.zeros((5, 28, 32), np.float32)
    di, i = np.meshgrid(np.arange(5), np.arange(28), indexing='ij')
    e1[di.ravel(), i.ravel(), (i + di).ravel()] = 1.0
    a1 = jnp.einsum('abde,dih->eaibh', conv1_w.astype(f32),
                    e1).reshape(5, 168, 96).astype(bf16)

    e2 = np.zeros((5, 10, 28), np.float32)
    di, i = np.meshgrid(np.arange(5), np.arange(10), indexing='ij')
    e2[di.ravel(), i.ravel(), (2 * (i + di)).ravel()] = 1.0
    a2 = jnp.einsum('abde,diu->eaibu', conv2_w.astype(f32),
                    e2).reshape(5, 160, 168).astype(bf16)

    g = np.zeros((400, 5, 160), np.float32)
    co2, pp, pw = np.meshgrid(np.arange(16), np.arange(5), np.arange(5),
                              indexing='ij')
    g[(25 * co2 + 5 * pp + pw).ravel(), pw.ravel(), (10 * co2 + 2 * pp).ravel()] = 1.0
    af = jnp.einsum('of,fpu->pou', fc1_w.astype(f32), g).astype(bf16)

    ones28 = np.ones((1, 28), np.float32)
    ones10 = np.ones((1, 10), np.float32)
    return dict(
        a1=a1, b1=(conv1_b.astype(f32)[:, None] * ones28).reshape(168, 1),
        a2=a2, b2=(conv2_b.astype(f32)[:, None] * ones10).reshape(160, 1),
        af=af, bf1=fc1_b.astype(f32).reshape(32, 1),
        w2=fc2_w.astype(bf16).T, bf2=fc2_b.astype(f32).reshape(1, 16),
        w3=fc3_w.astype(bf16).T, bf3=fc3_b.astype(f32).reshape(1, 10),
    )


def kernel(x, conv1_w, conv1_b, conv2_w, conv2_b,
           fc1_w, fc1_b, fc2_w, fc2_b, fc3_w, fc3_b):
    kp = _prep_params(conv1_w, conv1_b, conv2_w, conv2_b,
                      fc1_w, fc1_b, fc2_w, fc2_b, fc3_w, fc3_b)
    B = x.shape[0]
    Bp = ((B + _TB - 1) // _TB) * _TB
    xb = x.astype(jnp.bfloat16)
    if Bp != B:
        xb = jnp.pad(xb, ((0, Bp - B), (0, 0), (0, 0), (0, 0)))
    ntiles = Bp // _TB
    x3 = xb.reshape(Bp, 3, 1024)

    flops = int(Bp * 2 * (75 * 6 * 28 * 28 + 150 * 16 * 10 * 10
                          + 400 * 32 + 32 * 16 + 16 * 10))
    bytes_accessed = int(x3.size * 2 + Bp * 10 * 4
                         + sum(int(v.size) for v in kp.values()) * 2)

    out = pl.pallas_call(
        _net_kernel,
        out_shape=jax.ShapeDtypeStruct((Bp, 10), jnp.float32),
        grid=(ntiles,),
        in_specs=[
            pl.BlockSpec((_TB, 3, 1024), lambda i: (i, 0, 0)),
            pl.BlockSpec((5, 168, 96), lambda i: (0, 0, 0)),
            pl.BlockSpec((168, 1), lambda i: (0, 0)),
            pl.BlockSpec((5, 160, 168), lambda i: (0, 0, 0)),
            pl.BlockSpec((160, 1), lambda i: (0, 0)),
            pl.BlockSpec((5, 32, 160), lambda i: (0, 0, 0)),
            pl.BlockSpec((32, 1), lambda i: (0, 0)),
            pl.BlockSpec((32, 16), lambda i: (0, 0)),
            pl.BlockSpec((1, 16), lambda i: (0, 0)),
            pl.BlockSpec((16, 10), lambda i: (0, 0)),
            pl.BlockSpec((1, 10), lambda i: (0, 0)),
        ],
        out_specs=pl.BlockSpec((_TB, 10), lambda i: (i, 0)),
        compiler_params=pltpu.CompilerParams(dimension_semantics=("parallel",)),
        cost_estimate=pl.CostEstimate(flops=flops, transcendentals=0,
                                      bytes_accessed=bytes_accessed),
    )(x3, kp['a1'], kp['b1'], kp['a2'], kp['b2'], kp['af'], kp['bf1'],
      kp['w2'], kp['bf2'], kp['w3'], kp['bf3'])
    return out[:B]


# speedup vs baseline: 2.7774x; 1.1149x over previous
import functools

import numpy as np
import jax
import jax.numpy as jnp
from jax.experimental import pallas as pl
from jax.experimental.pallas import tpu as pltpu

_TB = 128
_N = 32 * _TB


def _shl(v, k):
    if k == 0:
        return v
    return jnp.concatenate([v[:, k:], v[:, :k]], axis=1)


def _shu(v):
    return jnp.concatenate([v[1:, :], v[:1, :]], axis=0)


def _net_kernel(x_ref, a1_ref, b1_ref, a2_ref, b2_ref, af_ref, bf1_ref,
                w2_ref, bf2_ref, w3_ref, bf3_ref, out_ref):
    pieces = []
    for c in range(3):
        tc = x_ref[:, c, :].T
        pieces.append(tc.reshape(32, _N).astype(jnp.bfloat16))
    x = jnp.concatenate(pieces, axis=0)
    acc = None
    for dj in range(5):
        d = jnp.dot(a1_ref[dj], _shl(x, dj * _TB),
                    preferred_element_type=jnp.float32)
        acc = d if acc is None else acc + d
    y1 = jnp.maximum(acc + b1_ref[...], 0.0)

    t = jnp.maximum(y1, _shl(y1, _TB))
    p1 = jnp.maximum(t, _shu(t)).astype(jnp.bfloat16)

    acc = None
    for dj in range(5):
        d = jnp.dot(a2_ref[dj], _shl(p1, 2 * dj * _TB),
                    preferred_element_type=jnp.float32)
        acc = d if acc is None else acc + d
    y2 = jnp.maximum(acc + b2_ref[...], 0.0)

    t = jnp.maximum(y2, _shl(y2, 2 * _TB))
    p2 = jnp.maximum(t, _shu(t)).astype(jnp.bfloat16)

    acc = None
    for pw in range(5):
        blk = p2[:, 4 * pw * _TB:(4 * pw + 1) * _TB]
        d = jnp.dot(af_ref[pw], blk, preferred_element_type=jnp.float32)
        acc = d if acc is None else acc + d
    h = jnp.maximum(acc + bf1_ref[...], 0.0)

    ht = h.T.astype(jnp.bfloat16)
    h2 = jnp.maximum(jnp.dot(ht, w2_ref[...], preferred_element_type=jnp.float32)
                     + bf2_ref[...], 0.0).astype(jnp.bfloat16)
    out = jnp.dot(h2, w3_ref[...], preferred_element_type=jnp.float32) + bf3_ref[...]
    out_ref[...] = out


def _prep_params(conv1_w, conv1_b, conv2_w, conv2_b,
                 fc1_w, fc1_b, fc2_w, fc2_b, fc3_w, fc3_b):
    f32, bf16 = jnp.float32, jnp.bfloat16

    e1 = np.zeros((5, 28, 32), np.float32)
    di, i = np.meshgrid(np.arange(5), np.arange(28), indexing='ij')
    e1[di.ravel(), i.ravel(), (i + di).ravel()] = 1.0
    a1 = jnp.einsum('abde,dih->eaibh', conv1_w.astype(f32),
                    e1).reshape(5, 168, 96).astype(bf16)

    e2 = np.zeros((5, 10, 28), np.float32)
    di, i = np.meshgrid(np.arange(5), np.arange(10), indexing='ij')
    e2[di.ravel(), i.ravel(), (2 * (i + di)).ravel()] = 1.0
    a2 = jnp.einsum('abde,diu->eaibu', conv2_w.astype(f32),
                    e2).reshape(5, 160, 168).astype(bf16)

    g = np.zeros((400, 5, 160), np.float32)
    co2, pp, pw = np.meshgrid(np.arange(16), np.arange(5), np.arange(5),
                              indexing='ij')
    g[(25 * co2 + 5 * pp + pw).ravel(), pw.ravel(), (10 * co2 + 2 * pp).ravel()] = 1.0
    af = jnp.einsum('of,fpu->pou', fc1_w.astype(f32), g).astype(bf16)

    ones28 = np.ones((1, 28), np.float32)
    ones10 = np.ones((1, 10), np.float32)
    return dict(
        a1=a1, b1=(conv1_b.astype(f32)[:, None] * ones28).reshape(168, 1),
        a2=a2, b2=(conv2_b.astype(f32)[:, None] * ones10).reshape(160, 1),
        af=af, bf1=fc1_b.astype(f32).reshape(32, 1),
        w2=fc2_w.astype(bf16).T, bf2=fc2_b.astype(f32).reshape(1, 16),
        w3=fc3_w.astype(bf16).T, bf3=fc3_b.astype(f32).reshape(1, 10),
    )


def kernel(x, conv1_w, conv1_b, conv2_w, conv2_b,
           fc1_w, fc1_b, fc2_w, fc2_b, fc3_w, fc3_b):
    kp = _prep_params(conv1_w, conv1_b, conv2_w, conv2_b,
                      fc1_w, fc1_b, fc2_w, fc2_b, fc3_w, fc3_b)
    B = x.shape[0]
    Bp = ((B + _TB - 1) // _TB) * _TB
    xb = x
    if Bp != B:
        xb = jnp.pad(xb, ((0, Bp - B), (0, 0), (0, 0), (0, 0)))
    ntiles = Bp // _TB
    x3 = xb.reshape(Bp, 3, 1024)

    flops = int(Bp * 2 * (75 * 6 * 28 * 28 + 150 * 16 * 10 * 10
                          + 400 * 32 + 32 * 16 + 16 * 10))
    bytes_accessed = int(x3.size * 4 + Bp * 10 * 4
                         + sum(int(v.size) for v in kp.values()) * 2)

    out = pl.pallas_call(
        _net_kernel,
        out_shape=jax.ShapeDtypeStruct((Bp, 10), jnp.float32),
        grid=(ntiles,),
        in_specs=[
            pl.BlockSpec((_TB, 3, 1024), lambda i: (i, 0, 0)),
            pl.BlockSpec((5, 168, 96), lambda i: (0, 0, 0)),
            pl.BlockSpec((168, 1), lambda i: (0, 0)),
            pl.BlockSpec((5, 160, 168), lambda i: (0, 0, 0)),
            pl.BlockSpec((160, 1), lambda i: (0, 0)),
            pl.BlockSpec((5, 32, 160), lambda i: (0, 0, 0)),
            pl.BlockSpec((32, 1), lambda i: (0, 0)),
            pl.BlockSpec((32, 16), lambda i: (0, 0)),
            pl.BlockSpec((1, 16), lambda i: (0, 0)),
            pl.BlockSpec((16, 10), lambda i: (0, 0)),
            pl.BlockSpec((1, 10), lambda i: (0, 0)),
        ],
        out_specs=pl.BlockSpec((_TB, 10), lambda i: (i, 0)),
        compiler_params=pltpu.CompilerParams(dimension_semantics=("parallel",)),
        cost_estimate=pl.CostEstimate(flops=flops, transcendentals=0,
                                      bytes_accessed=bytes_accessed),
    )(x3, kp['a1'], kp['b1'], kp['a2'], kp['b2'], kp['af'], kp['bf1'],
      kp['w2'], kp['bf2'], kp['w3'], kp['bf3'])
    return out[:B]


# speedup vs baseline: 3.2097x; 1.1556x over previous
import functools

import numpy as np
import jax
import jax.numpy as jnp
from jax.experimental import pallas as pl
from jax.experimental.pallas import tpu as pltpu

_TB = 256
_N = 32 * _TB


def _shl(v, k):
    if k == 0:
        return v
    return jnp.concatenate([v[:, k:], v[:, :k]], axis=1)


def _shu(v):
    return jnp.concatenate([v[1:, :], v[:1, :]], axis=0)


def _net_kernel(x_ref, a1_ref, b1_ref, a2_ref, b2_ref, af_ref, bf1_ref,
                w2_ref, bf2_ref, w3_ref, bf3_ref, out_ref):
    pieces = []
    for c in range(3):
        tc = x_ref[:, c, :].T
        pieces.append(tc.reshape(32, _N).astype(jnp.bfloat16))
    x = jnp.concatenate(pieces, axis=0)
    acc = None
    for dj in range(5):
        d = jnp.dot(a1_ref[dj], _shl(x, dj * _TB),
                    preferred_element_type=jnp.float32)
        acc = d if acc is None else acc + d
    y1 = jnp.maximum(acc + b1_ref[...], 0.0)

    t = jnp.maximum(y1, _shl(y1, _TB))
    p1 = jnp.maximum(t, _shu(t)).astype(jnp.bfloat16)

    acc = None
    for dj in range(5):
        d = jnp.dot(a2_ref[dj], _shl(p1, 2 * dj * _TB),
                    preferred_element_type=jnp.float32)
        acc = d if acc is None else acc + d
    y2 = jnp.maximum(acc + b2_ref[...], 0.0)

    t = jnp.maximum(y2, _shl(y2, 2 * _TB))
    p2 = jnp.maximum(t, _shu(t)).astype(jnp.bfloat16)

    acc = None
    for pw in range(5):
        blk = p2[:, 4 * pw * _TB:(4 * pw + 1) * _TB]
        d = jnp.dot(af_ref[pw], blk, preferred_element_type=jnp.float32)
        acc = d if acc is None else acc + d
    h = jnp.maximum(acc + bf1_ref[...], 0.0)

    ht = h.T.astype(jnp.bfloat16)
    h2 = jnp.maximum(jnp.dot(ht, w2_ref[...], preferred_element_type=jnp.float32)
                     + bf2_ref[...], 0.0).astype(jnp.bfloat16)
    out = jnp.dot(h2, w3_ref[...], preferred_element_type=jnp.float32) + bf3_ref[...]
    out_ref[...] = out


def _prep_params(conv1_w, conv1_b, conv2_w, conv2_b,
                 fc1_w, fc1_b, fc2_w, fc2_b, fc3_w, fc3_b):
    f32, bf16 = jnp.float32, jnp.bfloat16

    e1 = np.zeros((5, 28, 32), np.float32)
    di, i = np.meshgrid(np.arange(5), np.arange(28), indexing='ij')
    e1[di.ravel(), i.ravel(), (i + di).ravel()] = 1.0
    a1 = jnp.einsum('abde,dih->eaibh', conv1_w.astype(f32),
                    e1).reshape(5, 168, 96).astype(bf16)

    e2 = np.zeros((5, 10, 28), np.float32)
    di, i = np.meshgrid(np.arange(5), np.arange(10), indexing='ij')
    e2[di.ravel(), i.ravel(), (2 * (i + di)).ravel()] = 1.0
    a2 = jnp.einsum('abde,diu->eaibu', conv2_w.astype(f32),
                    e2).reshape(5, 160, 168).astype(bf16)

    g = np.zeros((400, 5, 160), np.float32)
    co2, pp, pw = np.meshgrid(np.arange(16), np.arange(5), np.arange(5),
                              indexing='ij')
    g[(25 * co2 + 5 * pp + pw).ravel(), pw.ravel(), (10 * co2 + 2 * pp).ravel()] = 1.0
    af = jnp.einsum('of,fpu->pou', fc1_w.astype(f32), g).astype(bf16)

    ones28 = np.ones((1, 28), np.float32)
    ones10 = np.ones((1, 10), np.float32)
    return dict(
        a1=a1, b1=(conv1_b.astype(f32)[:, None] * ones28).reshape(168, 1),
        a2=a2, b2=(conv2_b.astype(f32)[:, None] * ones10).reshape(160, 1),
        af=af, bf1=fc1_b.astype(f32).reshape(32, 1),
        w2=fc2_w.astype(bf16).T, bf2=fc2_b.astype(f32).reshape(1, 16),
        w3=fc3_w.astype(bf16).T, bf3=fc3_b.astype(f32).reshape(1, 10),
    )


def kernel(x, conv1_w, conv1_b, conv2_w, conv2_b,
           fc1_w, fc1_b, fc2_w, fc2_b, fc3_w, fc3_b):
    kp = _prep_params(conv1_w, conv1_b, conv2_w, conv2_b,
                      fc1_w, fc1_b, fc2_w, fc2_b, fc3_w, fc3_b)
    B = x.shape[0]
    Bp = ((B + _TB - 1) // _TB) * _TB
    xb = x
    if Bp != B:
        xb = jnp.pad(xb, ((0, Bp - B), (0, 0), (0, 0), (0, 0)))
    ntiles = Bp // _TB
    x3 = xb.reshape(Bp, 3, 1024)

    flops = int(Bp * 2 * (75 * 6 * 28 * 28 + 150 * 16 * 10 * 10
                          + 400 * 32 + 32 * 16 + 16 * 10))
    bytes_accessed = int(x3.size * 4 + Bp * 10 * 4
                         + sum(int(v.size) for v in kp.values()) * 2)

    out = pl.pallas_call(
        _net_kernel,
        out_shape=jax.ShapeDtypeStruct((Bp, 10), jnp.float32),
        grid=(ntiles,),
        in_specs=[
            pl.BlockSpec((_TB, 3, 1024), lambda i: (i, 0, 0)),
            pl.BlockSpec((5, 168, 96), lambda i: (0, 0, 0)),
            pl.BlockSpec((168, 1), lambda i: (0, 0)),
            pl.BlockSpec((5, 160, 168), lambda i: (0, 0, 0)),
            pl.BlockSpec((160, 1), lambda i: (0, 0)),
            pl.BlockSpec((5, 32, 160), lambda i: (0, 0, 0)),
            pl.BlockSpec((32, 1), lambda i: (0, 0)),
            pl.BlockSpec((32, 16), lambda i: (0, 0)),
            pl.BlockSpec((1, 16), lambda i: (0, 0)),
            pl.BlockSpec((16, 10), lambda i: (0, 0)),
            pl.BlockSpec((1, 10), lambda i: (0, 0)),
        ],
        out_specs=pl.BlockSpec((_TB, 10), lambda i: (i, 0)),
        compiler_params=pltpu.CompilerParams(dimension_semantics=("parallel",)),
        cost_estimate=pl.CostEstimate(flops=flops, transcendentals=0,
                                      bytes_accessed=bytes_accessed),
    )(x3, kp['a1'], kp['b1'], kp['a2'], kp['b2'], kp['af'], kp['bf1'],
      kp['w2'], kp['bf2'], kp['w3'], kp['bf3'])
    return out[:B]
